# baseline (speedup 1.0000x reference)
import functools

import jax
import jax.numpy as jnp
import numpy as np
from jax import lax
from jax.experimental import pallas as pl
from jax.experimental.pallas import tpu as pltpu
from jax.experimental.pallas import tpu_sc as plsc

_B, _K, _H, _W = 64, 17, 128, 128
_V = _H * _W
_G = 8
_R = _B * _K
_ROWS = 16
_NW = 32
_X_SC = 336

_TINY = np.float32(np.finfo(np.float32).tiny)
_EPS = np.float32(1e-8)
_K1 = np.int32(42)
_K2 = np.int32(0 ^ 42 ^ 0x1BD11BDA)
_ROT_A = (13, 15, 26, 6)
_ROT_B = (17, 29, 16, 24)


def _rotl(x, d):
    return lax.shift_left(x, np.int32(d)) | lax.shift_right_logical(
        x, np.int32(32 - d)
    )


def _four_rounds(x0, x1, rots):
    for r in rots:
        x0 = x0 + x1
        x1 = _rotl(x1, r)
        x1 = x0 ^ x1
    return x0, x1


def _threefry_bits(x1):
    x0 = x1
    x1 = x0 ^ _rotl(x1, _ROT_A[0])
    for r in _ROT_A[1:]:
        x0 = x0 + x1
        x1 = _rotl(x1, r)
        x1 = x0 ^ x1
    x0 = x0 + _K1
    x1 = x1 + np.int32(_K2 + 1)
    x0, x1 = _four_rounds(x0, x1, _ROT_B)
    x0 = x0 + _K2
    x1 = x1 + np.int32(2)
    x0, x1 = _four_rounds(x0, x1, _ROT_A)
    x1 = x1 + np.int32(_K1 + 3)
    x0, x1 = _four_rounds(x0, x1, _ROT_B)
    x0 = x0 + _K1
    x1 = x1 + np.int32(_K2 + 4)
    x0, x1 = _four_rounds(x0, x1, _ROT_A)
    x0 = x0 + _K2
    x1 = x1 + np.int32(5)
    return x0 ^ x1


def _bits_to_u(bits):
    fbits = lax.shift_right_logical(bits, np.int32(9)) | np.int32(0x3F800000)
    f = lax.bitcast_convert_type(fbits, jnp.float32) - np.float32(1.0)
    return jnp.maximum(_TINY, f + _TINY)


def _sc_body(out_hbm, buf, sem):
    c = lax.axis_index("c")
    s_ax = lax.axis_index("s")
    wid = s_ax * np.int32(2) + c
    ppw = np.int32(_X_SC * _G // _NW)
    lane = lax.iota(jnp.int32, 16)

    def pair_body(i, carry):
        pair = wid * ppw + i
        row = pair // np.int32(_G)
        s = pair % np.int32(_G)
        base = (s * np.int32(_R) + row) * np.int32(_V) + np.int32(42)

        def chunk_body(cc, carry2):
            h = cc
            for q in range(8):
                p = base + h * np.int32(_W) + np.int32(q * 16) + lane
                buf[h, pl.ds(q * 16, 16)] = _bits_to_u(_threefry_bits(p))
            return carry2

        lax.fori_loop(0, _H, chunk_body, 0, unroll=2)
        pltpu.sync_copy(buf, out_hbm.at[row, s])
        return carry

    lax.fori_loop(0, ppw, pair_body, 0, unroll=False)


def _sc_uniforms():
    mesh = plsc.VectorSubcoreMesh(core_axis_name="c", subcore_axis_name="s")
    return pl.kernel(
        _sc_body,
        mesh=mesh,
        out_type=jax.ShapeDtypeStruct((_X_SC, _G, _H, _W), jnp.float32),
        scratch_types=[
            pltpu.VMEM((_H, _W), jnp.float32),
            pltpu.SemaphoreType.DMA,
        ],
    )()


def _extract(l, vi, z, m, lse):
    zm = jnp.max(z)
    win = jnp.min(jnp.where(z == zm, vi, np.int32(_V)))
    lwin = jnp.sum(jnp.where(vi == win, l, np.float32(0.0)))
    return win, (lwin - m) - lse


def _row_outputs(wins, logps, rr, sub, lane, idx_out, logp_out):
    for s in range(_G):
        here = (sub == rr) & (lane == s)
        idx_out = jnp.where(here, wins[s], idx_out)
        logp_out = jnp.where(here, logps[s], logp_out)
    return idx_out, logp_out


def _sample_body(hm_ref, idx_ref, logp_ref, *, row_offset):
    rb = pl.program_id(0)
    vi = (
        lax.broadcasted_iota(jnp.int32, (_H, _W), 0) * np.int32(_W)
        + lax.broadcasted_iota(jnp.int32, (_H, _W), 1)
    )
    sub = lax.broadcasted_iota(jnp.int32, (_ROWS, 1, _G), 0)
    lane = lax.broadcasted_iota(jnp.int32, (_ROWS, 1, _G), 2)
    idx_out = jnp.zeros((_ROWS, 1, _G), jnp.int32)
    logp_out = jnp.zeros((_ROWS, 1, _G), jnp.float32)

    for rr in range(_ROWS):
        r = rb * np.int32(_ROWS) + np.int32(row_offset + rr)
        l = hm_ref[rr]
        m = jnp.max(l)
        lse = jnp.log(jnp.sum(jnp.exp(l - m)))
        zs = []
        for s in range(_G):
            base = (np.int32(s * _R) + r) * np.int32(_V) + np.int32(42)
            u = _bits_to_u(_threefry_bits(base + vi))
            zs.append(-jnp.log(-jnp.log(u)) + l)
        zms = [jnp.max(z) for z in zs]
        wins = []
        logps = []
        for s in range(_G):
            win = jnp.min(jnp.where(zs[s] == zms[s], vi, np.int32(_V)))
            wins.append(win)
        for s in range(_G):
            lwin = jnp.sum(jnp.where(vi == wins[s], l, np.float32(0.0)))
            logps.append((lwin - m) - lse)
        idx_out, logp_out = _row_outputs(
            wins, logps, rr, sub, lane, idx_out, logp_out
        )
    idx_ref[...] = idx_out
    logp_ref[...] = logp_out


def _assist_body(hm_ref, u_ref, idx_ref, logp_ref):
    vi = (
        lax.broadcasted_iota(jnp.int32, (_H, _W), 0) * np.int32(_W)
        + lax.broadcasted_iota(jnp.int32, (_H, _W), 1)
    )
    sub = lax.broadcasted_iota(jnp.int32, (_ROWS, 1, _G), 0)
    lane = lax.broadcasted_iota(jnp.int32, (_ROWS, 1, _G), 2)
    idx_out = jnp.zeros((_ROWS, 1, _G), jnp.int32)
    logp_out = jnp.zeros((_ROWS, 1, _G), jnp.float32)

    for rr in range(_ROWS):
        l = hm_ref[rr]
        m = jnp.max(l)
        lse = jnp.log(jnp.sum(jnp.exp(l - m)))
        zs = [
            -jnp.log(-jnp.log(u_ref[rr, s])) + l for s in range(_G)
        ]
        zms = [jnp.max(z) for z in zs]
        wins = []
        logps = []
        for s in range(_G):
            win = jnp.min(jnp.where(zs[s] == zms[s], vi, np.int32(_V)))
            wins.append(win)
        for s in range(_G):
            lwin = jnp.sum(jnp.where(vi == wins[s], l, np.float32(0.0)))
            logps.append((lwin - m) - lse)
        idx_out, logp_out = _row_outputs(
            wins, logps, rr, sub, lane, idx_out, logp_out
        )
    idx_ref[...] = idx_out
    logp_ref[...] = logp_out


def _loss_body(idx_ref, logp_ref, out_ref):
    idx = idx_ref[...]
    logp = logp_ref[...]
    x = (idx % np.int32(_W)).astype(jnp.float32)
    y = (idx // np.int32(_W)).astype(jnp.float32)
    cx = np.float32((_W - 1) / 2.0)
    cy = np.float32((_H - 1) / 2.0)
    d = jnp.sqrt((x - cx) * (x - cx) + (y - cy) * (y - cy))
    rewards = -(jnp.sum(d, axis=1) / np.float32(_K)) / np.float32(max(_H, _W))
    rmean = jnp.mean(rewards, axis=-1, keepdims=True)
    dev = rewards - rmean
    std = jnp.sqrt(jnp.sum(dev * dev, axis=-1, keepdims=True) / np.float32(_G - 1))
    adv = dev / jnp.maximum(std, _EPS)
    adv = jnp.clip(adv, -5.0, 5.0)
    log_pi = jnp.sum(logp, axis=1)
    loss = -jnp.mean(adv * log_pi)
    reward_mean = jnp.mean(rewards)
    rdev = rewards - reward_mean
    reward_std = jnp.sqrt(jnp.sum(rdev * rdev) / np.float32(_B * _G - 1))
    adv_abs_mean = jnp.mean(jnp.abs(adv))
    lanes = lax.broadcasted_iota(jnp.int32, (1, 128), 1)
    vec = jnp.where(lanes == 0, loss, np.float32(0.0))
    vec = jnp.where(lanes == 1, reward_mean, vec)
    vec = jnp.where(lanes == 2, reward_std, vec)
    vec = jnp.where(lanes == 3, adv_abs_mean, vec)
    out_ref[...] = vec


def _run(heatmaps, interpret=False):
    hm = heatmaps.reshape(_R, _H, _W)

    u_sc = _sc_uniforms()

    n_main = _R - _X_SC
    idx_m, logp_m = pl.pallas_call(
        functools.partial(_sample_body, row_offset=_X_SC),
        grid=(n_main // _ROWS,),
        in_specs=[
            pl.BlockSpec((_ROWS, _H, _W), lambda r: (r + _X_SC // _ROWS, 0, 0)),
        ],
        out_specs=[
            pl.BlockSpec((_ROWS, 1, _G), lambda r: (r, 0, 0)),
            pl.BlockSpec((_ROWS, 1, _G), lambda r: (r, 0, 0)),
        ],
        out_shape=[
            jax.ShapeDtypeStruct((n_main, 1, _G), jnp.int32),
            jax.ShapeDtypeStruct((n_main, 1, _G), jnp.float32),
        ],
        compiler_params=pltpu.CompilerParams(
            dimension_semantics=("parallel",)
        ),
        interpret=interpret,
    )(hm)

    idx_a, logp_a = pl.pallas_call(
        _assist_body,
        grid=(_X_SC // _ROWS,),
        in_specs=[
            pl.BlockSpec((_ROWS, _H, _W), lambda r: (r, 0, 0)),
            pl.BlockSpec((_ROWS, _G, _H, _W), lambda r: (r, 0, 0, 0)),
        ],
        out_specs=[
            pl.BlockSpec((_ROWS, 1, _G), lambda r: (r, 0, 0)),
            pl.BlockSpec((_ROWS, 1, _G), lambda r: (r, 0, 0)),
        ],
        out_shape=[
            jax.ShapeDtypeStruct((_X_SC, 1, _G), jnp.int32),
            jax.ShapeDtypeStruct((_X_SC, 1, _G), jnp.float32),
        ],
        compiler_params=pltpu.CompilerParams(
            dimension_semantics=("parallel",)
        ),
        interpret=interpret,
    )(hm[:_X_SC], u_sc)

    idx = jnp.concatenate([idx_a, idx_m], axis=0).reshape(_B, _K, _G)
    logp = jnp.concatenate([logp_a, logp_m], axis=0).reshape(_B, _K, _G)
    out = pl.pallas_call(
        _loss_body,
        in_specs=[
            pl.BlockSpec(idx.shape, lambda: (0, 0, 0)),
            pl.BlockSpec(logp.shape, lambda: (0, 0, 0)),
        ],
        out_specs=pl.BlockSpec((1, 128), lambda: (0, 0)),
        out_shape=jax.ShapeDtypeStruct((1, 128), jnp.float32),
        interpret=interpret,
    )(idx, logp)
    return (out[0, 0], out[0, 1], out[0, 2], out[0, 3])


def kernel(heatmaps):
    return _run(heatmaps)


# speedup vs baseline: 1.1925x; 1.1925x over previous
import functools

import jax
import jax.numpy as jnp
import numpy as np
from jax import lax
from jax.experimental import pallas as pl
from jax.experimental.pallas import tpu as pltpu
from jax.experimental.pallas import tpu_sc as plsc

_B, _K, _H, _W = 64, 17, 128, 128
_V = _H * _W
_G = 8
_R = _B * _K
_ROWS = 8
_NW = 32
_X_SC = 336

_TINY = np.float32(np.finfo(np.float32).tiny)
_EPS = np.float32(1e-8)
_K1 = np.int32(42)
_K2 = np.int32(0 ^ 42 ^ 0x1BD11BDA)
_ROT_A = (13, 15, 26, 6)
_ROT_B = (17, 29, 16, 24)


def _rotl(x, d):
    return lax.shift_left(x, np.int32(d)) | lax.shift_right_logical(
        x, np.int32(32 - d)
    )


def _four_rounds(x0, x1, rots):
    for r in rots:
        x0 = x0 + x1
        x1 = _rotl(x1, r)
        x1 = x0 ^ x1
    return x0, x1


def _threefry_bits(x1):
    x0 = x1
    x1 = x0 ^ _rotl(x1, _ROT_A[0])
    for r in _ROT_A[1:]:
        x0 = x0 + x1
        x1 = _rotl(x1, r)
        x1 = x0 ^ x1
    x0 = x0 + _K1
    x1 = x1 + np.int32(_K2 + 1)
    x0, x1 = _four_rounds(x0, x1, _ROT_B)
    x0 = x0 + _K2
    x1 = x1 + np.int32(2)
    x0, x1 = _four_rounds(x0, x1, _ROT_A)
    x1 = x1 + np.int32(_K1 + 3)
    x0, x1 = _four_rounds(x0, x1, _ROT_B)
    x0 = x0 + _K1
    x1 = x1 + np.int32(_K2 + 4)
    x0, x1 = _four_rounds(x0, x1, _ROT_A)
    x0 = x0 + _K2
    x1 = x1 + np.int32(5)
    return x0 ^ x1


def _bits_to_u(bits):
    fbits = lax.shift_right_logical(bits, np.int32(9)) | np.int32(0x3F800000)
    f = lax.bitcast_convert_type(fbits, jnp.float32) - np.float32(1.0)
    return jnp.maximum(_TINY, f + _TINY)


def _sc_body(out_hbm, buf, sem):
    c = lax.axis_index("c")
    s_ax = lax.axis_index("s")
    wid = s_ax * np.int32(2) + c
    ppw = np.int32(_X_SC * _G // _NW)
    lane = lax.iota(jnp.int32, 16)

    def pair_body(i, carry):
        pair = wid * ppw + i
        row = pair // np.int32(_G)
        s = pair % np.int32(_G)
        base = (s * np.int32(_R) + row) * np.int32(_V) + np.int32(42)

        def chunk_body(cc, carry2):
            h = cc
            for q in range(8):
                p = base + h * np.int32(_W) + np.int32(q * 16) + lane
                buf[h, pl.ds(q * 16, 16)] = _bits_to_u(_threefry_bits(p))
            return carry2

        lax.fori_loop(0, _H, chunk_body, 0, unroll=2)
        pltpu.sync_copy(buf, out_hbm.at[row, s])
        return carry

    lax.fori_loop(0, ppw, pair_body, 0, unroll=False)


def _sc_uniforms():
    mesh = plsc.VectorSubcoreMesh(core_axis_name="c", subcore_axis_name="s")
    return pl.kernel(
        _sc_body,
        mesh=mesh,
        out_type=jax.ShapeDtypeStruct((_X_SC, _G, _H, _W), jnp.float32),
        scratch_types=[
            pltpu.VMEM((_H, _W), jnp.float32),
            pltpu.SemaphoreType.DMA,
        ],
    )()


def _extract(l, vi, z, m, lse):
    zm = jnp.max(z)
    win = jnp.min(jnp.where(z == zm, vi, np.int32(_V)))
    lwin = jnp.sum(jnp.where(vi == win, l, np.float32(0.0)))
    return win, (lwin - m) - lse


def _row_outputs(wins, logps, rr, sub, lane, idx_out, logp_out):
    for s in range(_G):
        here = (sub == rr) & (lane == s)
        idx_out = jnp.where(here, wins[s], idx_out)
        logp_out = jnp.where(here, logps[s], logp_out)
    return idx_out, logp_out


def _sample_body(hm_ref, idx_ref, logp_ref, *, row_offset):
    rb = pl.program_id(0)
    vi = (
        lax.broadcasted_iota(jnp.int32, (_H, _W), 0) * np.int32(_W)
        + lax.broadcasted_iota(jnp.int32, (_H, _W), 1)
    )
    sub = lax.broadcasted_iota(jnp.int32, (_ROWS, 1, _G), 0)
    lane = lax.broadcasted_iota(jnp.int32, (_ROWS, 1, _G), 2)
    idx_out = jnp.zeros((_ROWS, 1, _G), jnp.int32)
    logp_out = jnp.zeros((_ROWS, 1, _G), jnp.float32)

    for rr in range(_ROWS):
        r = rb * np.int32(_ROWS) + np.int32(row_offset + rr)
        l = hm_ref[rr]
        m = jnp.max(l)
        lse = jnp.log(jnp.sum(jnp.exp(l - m)))
        zs = []
        for s in range(_G):
            base = (np.int32(s * _R) + r) * np.int32(_V) + np.int32(42)
            u = _bits_to_u(_threefry_bits(base + vi))
            zs.append(-jnp.log(-jnp.log(u)) + l)
        zms = [jnp.max(z) for z in zs]
        wins = []
        logps = []
        for s in range(_G):
            win = jnp.min(jnp.where(zs[s] == zms[s], vi, np.int32(_V)))
            wins.append(win)
        for s in range(_G):
            lwin = jnp.sum(jnp.where(vi == wins[s], l, np.float32(0.0)))
            logps.append((lwin - m) - lse)
        idx_out, logp_out = _row_outputs(
            wins, logps, rr, sub, lane, idx_out, logp_out
        )
    idx_ref[...] = idx_out
    logp_ref[...] = logp_out


def _assist_body(hm_ref, u_ref, idx_ref, logp_ref):
    vi = (
        lax.broadcasted_iota(jnp.int32, (_H, _W), 0) * np.int32(_W)
        + lax.broadcasted_iota(jnp.int32, (_H, _W), 1)
    )
    sub = lax.broadcasted_iota(jnp.int32, (_ROWS, 1, _G), 0)
    lane = lax.broadcasted_iota(jnp.int32, (_ROWS, 1, _G), 2)
    idx_out = jnp.zeros((_ROWS, 1, _G), jnp.int32)
    logp_out = jnp.zeros((_ROWS, 1, _G), jnp.float32)

    for rr in range(_ROWS):
        l = hm_ref[rr]
        m = jnp.max(l)
        lse = jnp.log(jnp.sum(jnp.exp(l - m)))
        zs = [
            -jnp.log(-jnp.log(u_ref[rr, s])) + l for s in range(_G)
        ]
        zms = [jnp.max(z) for z in zs]
        wins = []
        logps = []
        for s in range(_G):
            win = jnp.min(jnp.where(zs[s] == zms[s], vi, np.int32(_V)))
            wins.append(win)
        for s in range(_G):
            lwin = jnp.sum(jnp.where(vi == wins[s], l, np.float32(0.0)))
            logps.append((lwin - m) - lse)
        idx_out, logp_out = _row_outputs(
            wins, logps, rr, sub, lane, idx_out, logp_out
        )
    idx_ref[...] = idx_out
    logp_ref[...] = logp_out


def _loss_body(idx_ref, logp_ref, out_ref):
    idx = idx_ref[...]
    logp = logp_ref[...]
    x = (idx % np.int32(_W)).astype(jnp.float32)
    y = (idx // np.int32(_W)).astype(jnp.float32)
    cx = np.float32((_W - 1) / 2.0)
    cy = np.float32((_H - 1) / 2.0)
    d = jnp.sqrt((x - cx) * (x - cx) + (y - cy) * (y - cy))
    rewards = -(jnp.sum(d, axis=1) / np.float32(_K)) / np.float32(max(_H, _W))
    rmean = jnp.mean(rewards, axis=-1, keepdims=True)
    dev = rewards - rmean
    std = jnp.sqrt(jnp.sum(dev * dev, axis=-1, keepdims=True) / np.float32(_G - 1))
    adv = dev / jnp.maximum(std, _EPS)
    adv = jnp.clip(adv, -5.0, 5.0)
    log_pi = jnp.sum(logp, axis=1)
    loss = -jnp.mean(adv * log_pi)
    reward_mean = jnp.mean(rewards)
    rdev = rewards - reward_mean
    reward_std = jnp.sqrt(jnp.sum(rdev * rdev) / np.float32(_B * _G - 1))
    adv_abs_mean = jnp.mean(jnp.abs(adv))
    lanes = lax.broadcasted_iota(jnp.int32, (1, 128), 1)
    vec = jnp.where(lanes == 0, loss, np.float32(0.0))
    vec = jnp.where(lanes == 1, reward_mean, vec)
    vec = jnp.where(lanes == 2, reward_std, vec)
    vec = jnp.where(lanes == 3, adv_abs_mean, vec)
    out_ref[...] = vec


def _run(heatmaps, interpret=False):
    hm = heatmaps.reshape(_R, _H, _W)

    u_sc = _sc_uniforms()

    n_main = _R - _X_SC
    idx_m, logp_m = pl.pallas_call(
        functools.partial(_sample_body, row_offset=_X_SC),
        grid=(n_main // _ROWS,),
        in_specs=[
            pl.BlockSpec((_ROWS, _H, _W), lambda r: (r + _X_SC // _ROWS, 0, 0)),
        ],
        out_specs=[
            pl.BlockSpec((_ROWS, 1, _G), lambda r: (r, 0, 0)),
            pl.BlockSpec((_ROWS, 1, _G), lambda r: (r, 0, 0)),
        ],
        out_shape=[
            jax.ShapeDtypeStruct((n_main, 1, _G), jnp.int32),
            jax.ShapeDtypeStruct((n_main, 1, _G), jnp.float32),
        ],
        compiler_params=pltpu.CompilerParams(
            dimension_semantics=("parallel",)
        ),
        interpret=interpret,
    )(hm)

    idx_a, logp_a = pl.pallas_call(
        _assist_body,
        grid=(_X_SC // _ROWS,),
        in_specs=[
            pl.BlockSpec((_ROWS, _H, _W), lambda r: (r, 0, 0)),
            pl.BlockSpec((_ROWS, _G, _H, _W), lambda r: (r, 0, 0, 0)),
        ],
        out_specs=[
            pl.BlockSpec((_ROWS, 1, _G), lambda r: (r, 0, 0)),
            pl.BlockSpec((_ROWS, 1, _G), lambda r: (r, 0, 0)),
        ],
        out_shape=[
            jax.ShapeDtypeStruct((_X_SC, 1, _G), jnp.int32),
            jax.ShapeDtypeStruct((_X_SC, 1, _G), jnp.float32),
        ],
        compiler_params=pltpu.CompilerParams(
            dimension_semantics=("parallel",)
        ),
        interpret=interpret,
    )(hm[:_X_SC], u_sc)

    idx = jnp.concatenate([idx_a, idx_m], axis=0).reshape(_B, _K, _G)
    logp = jnp.concatenate([logp_a, logp_m], axis=0).reshape(_B, _K, _G)
    out = pl.pallas_call(
        _loss_body,
        in_specs=[
            pl.BlockSpec(idx.shape, lambda: (0, 0, 0)),
            pl.BlockSpec(logp.shape, lambda: (0, 0, 0)),
        ],
        out_specs=pl.BlockSpec((1, 128), lambda: (0, 0)),
        out_shape=jax.ShapeDtypeStruct((1, 128), jnp.float32),
        interpret=interpret,
    )(idx, logp)
    return (out[0, 0], out[0, 1], out[0, 2], out[0, 3])


def kernel(heatmaps):
    return _run(heatmaps)


# speedup vs baseline: 1.1932x; 1.0006x over previous
import functools

import jax
import jax.numpy as jnp
import numpy as np
from jax import lax
from jax.experimental import pallas as pl
from jax.experimental.pallas import tpu as pltpu
from jax.experimental.pallas import tpu_sc as plsc

_B, _K, _H, _W = 64, 17, 128, 128
_V = _H * _W
_G = 8
_R = _B * _K
_ROWS = 8
_NW = 32
_X_SC = 336

_TINY = np.float32(np.finfo(np.float32).tiny)
_EPS = np.float32(1e-8)
_K1 = np.int32(42)
_K2 = np.int32(0 ^ 42 ^ 0x1BD11BDA)
_ROT_A = (13, 15, 26, 6)
_ROT_B = (17, 29, 16, 24)


def _rotl(x, d):
    return lax.shift_left(x, np.int32(d)) | lax.shift_right_logical(
        x, np.int32(32 - d)
    )


def _four_rounds(x0, x1, rots):
    for r in rots:
        x0 = x0 + x1
        x1 = _rotl(x1, r)
        x1 = x0 ^ x1
    return x0, x1


def _threefry_bits(x1):
    x0 = x1
    x1 = x0 ^ _rotl(x1, _ROT_A[0])
    for r in _ROT_A[1:]:
        x0 = x0 + x1
        x1 = _rotl(x1, r)
        x1 = x0 ^ x1
    x0 = x0 + _K1
    x1 = x1 + np.int32(_K2 + 1)
    x0, x1 = _four_rounds(x0, x1, _ROT_B)
    x0 = x0 + _K2
    x1 = x1 + np.int32(2)
    x0, x1 = _four_rounds(x0, x1, _ROT_A)
    x1 = x1 + np.int32(_K1 + 3)
    x0, x1 = _four_rounds(x0, x1, _ROT_B)
    x0 = x0 + _K1
    x1 = x1 + np.int32(_K2 + 4)
    x0, x1 = _four_rounds(x0, x1, _ROT_A)
    x0 = x0 + _K2
    x1 = x1 + np.int32(5)
    return x0 ^ x1


def _bits_to_u(bits):
    fbits = lax.shift_right_logical(bits, np.int32(9)) | np.int32(0x3F800000)
    f = lax.bitcast_convert_type(fbits, jnp.float32) - np.float32(1.0)
    return jnp.maximum(_TINY, f + _TINY)


def _sc_body(out_hbm, buf, sem):
    c = lax.axis_index("c")
    s_ax = lax.axis_index("s")
    wid = s_ax * np.int32(2) + c
    ppw = np.int32(_X_SC * _G // _NW)
    lane = lax.iota(jnp.int32, 16)

    def pair_body(i, carry):
        pair = wid * ppw + i
        row = pair // np.int32(_G)
        s = pair % np.int32(_G)
        base = (s * np.int32(_R) + row) * np.int32(_V) + np.int32(42)

        def chunk_body(cc, carry2):
            h = cc
            for q in range(8):
                p = base + h * np.int32(_W) + np.int32(q * 16) + lane
                buf[h, pl.ds(q * 16, 16)] = _bits_to_u(_threefry_bits(p))
            return carry2

        lax.fori_loop(0, _H, chunk_body, 0, unroll=2)
        pltpu.sync_copy(buf, out_hbm.at[row, s])
        return carry

    lax.fori_loop(0, ppw, pair_body, 0, unroll=False)


def _sc_uniforms():
    mesh = plsc.VectorSubcoreMesh(core_axis_name="c", subcore_axis_name="s")
    return pl.kernel(
        _sc_body,
        mesh=mesh,
        out_type=jax.ShapeDtypeStruct((_X_SC, _G, _H, _W), jnp.float32),
        scratch_types=[
            pltpu.VMEM((_H, _W), jnp.float32),
            pltpu.SemaphoreType.DMA,
        ],
    )()


def _extract(l, vi, z, m, lse):
    zm = jnp.max(z)
    win = jnp.min(jnp.where(z == zm, vi, np.int32(_V)))
    lwin = jnp.sum(jnp.where(vi == win, l, np.float32(0.0)))
    return win, (lwin - m) - lse


def _row_outputs(wins, logps, rr, sub, lane, idx_out, logp_out):
    for s in range(_G):
        here = (sub == rr) & (lane == s)
        idx_out = jnp.where(here, wins[s], idx_out)
        logp_out = jnp.where(here, logps[s], logp_out)
    return idx_out, logp_out


def _sample_body(hm_ref, idx_ref, logp_ref, *, row_offset):
    rb = pl.program_id(0)
    vi = (
        lax.broadcasted_iota(jnp.int32, (_H, _W), 0) * np.int32(_W)
        + lax.broadcasted_iota(jnp.int32, (_H, _W), 1)
    )
    sub = lax.broadcasted_iota(jnp.int32, (_ROWS, 1, _G), 0)
    lane = lax.broadcasted_iota(jnp.int32, (_ROWS, 1, _G), 2)
    idx_out = jnp.zeros((_ROWS, 1, _G), jnp.int32)
    logp_out = jnp.zeros((_ROWS, 1, _G), jnp.float32)

    for rr in range(_ROWS):
        r = rb * np.int32(_ROWS) + np.int32(row_offset + rr)
        l = hm_ref[rr]
        m = jnp.max(l)
        lse = jnp.log(jnp.sum(jnp.exp(l - m)))
        zs = []
        for s in range(_G):
            base = (np.int32(s * _R) + r) * np.int32(_V) + np.int32(42)
            u = _bits_to_u(_threefry_bits(base + vi))
            zs.append(-jnp.log(-jnp.log(u)) + l)
        zms = [jnp.max(z) for z in zs]
        wins = []
        logps = []
        for s in range(_G):
            win = jnp.min(jnp.where(zs[s] == zms[s], vi, np.int32(_V)))
            wins.append(win)
        for s in range(_G):
            lwin = jnp.sum(jnp.where(vi == wins[s], l, np.float32(0.0)))
            logps.append((lwin - m) - lse)
        idx_out, logp_out = _row_outputs(
            wins, logps, rr, sub, lane, idx_out, logp_out
        )
    idx_ref[...] = idx_out
    logp_ref[...] = logp_out


def _assist_body(hm_ref, u_ref, idx_ref, logp_ref):
    vi = (
        lax.broadcasted_iota(jnp.int32, (_H, _W), 0) * np.int32(_W)
        + lax.broadcasted_iota(jnp.int32, (_H, _W), 1)
    )
    sub = lax.broadcasted_iota(jnp.int32, (_ROWS, 1, _G), 0)
    lane = lax.broadcasted_iota(jnp.int32, (_ROWS, 1, _G), 2)
    idx_out = jnp.zeros((_ROWS, 1, _G), jnp.int32)
    logp_out = jnp.zeros((_ROWS, 1, _G), jnp.float32)

    for rr in range(_ROWS):
        l = hm_ref[rr]
        m = jnp.max(l)
        lse = jnp.log(jnp.sum(jnp.exp(l - m)))
        zs = [
            -jnp.log(-jnp.log(u_ref[rr, s])) + l for s in range(_G)
        ]
        zms = [jnp.max(z) for z in zs]
        wins = []
        logps = []
        for s in range(_G):
            win = jnp.min(jnp.where(zs[s] == zms[s], vi, np.int32(_V)))
            wins.append(win)
        for s in range(_G):
            lwin = jnp.sum(jnp.where(vi == wins[s], l, np.float32(0.0)))
            logps.append((lwin - m) - lse)
        idx_out, logp_out = _row_outputs(
            wins, logps, rr, sub, lane, idx_out, logp_out
        )
    idx_ref[...] = idx_out
    logp_ref[...] = logp_out


def _loss_body(idx_ref, logp_ref, out_ref):
    idx = idx_ref[...]
    logp = logp_ref[...]
    x = (idx % np.int32(_W)).astype(jnp.float32)
    y = (idx // np.int32(_W)).astype(jnp.float32)
    cx = np.float32((_W - 1) / 2.0)
    cy = np.float32((_H - 1) / 2.0)
    d = jnp.sqrt((x - cx) * (x - cx) + (y - cy) * (y - cy))
    rewards = -(jnp.sum(d, axis=1) / np.float32(_K)) / np.float32(max(_H, _W))
    rmean = jnp.mean(rewards, axis=-1, keepdims=True)
    dev = rewards - rmean
    std = jnp.sqrt(jnp.sum(dev * dev, axis=-1, keepdims=True) / np.float32(_G - 1))
    adv = dev / jnp.maximum(std, _EPS)
    adv = jnp.clip(adv, -5.0, 5.0)
    log_pi = jnp.sum(logp, axis=1)
    loss = -jnp.mean(adv * log_pi)
    reward_mean = jnp.mean(rewards)
    rdev = rewards - reward_mean
    reward_std = jnp.sqrt(jnp.sum(rdev * rdev) / np.float32(_B * _G - 1))
    adv_abs_mean = jnp.mean(jnp.abs(adv))
    lanes = lax.broadcasted_iota(jnp.int32, (1, 128), 1)
    vec = jnp.where(lanes == 0, loss, np.float32(0.0))
    vec = jnp.where(lanes == 1, reward_mean, vec)
    vec = jnp.where(lanes == 2, reward_std, vec)
    vec = jnp.where(lanes == 3, adv_abs_mean, vec)
    out_ref[...] = vec


def _run(heatmaps, interpret=False):
    hm = heatmaps.reshape(_R, _H, _W)

    u_sc = _sc_uniforms()

    n_main = _R - _X_SC
    idx_m, logp_m = pl.pallas_call(
        functools.partial(_sample_body, row_offset=_X_SC),
        grid=(n_main // _ROWS,),
        in_specs=[
            pl.BlockSpec((_ROWS, _H, _W), lambda r: (r + _X_SC // _ROWS, 0, 0)),
        ],
        out_specs=[
            pl.BlockSpec((_ROWS, 1, _G), lambda r: (r, 0, 0)),
            pl.BlockSpec((_ROWS, 1, _G), lambda r: (r, 0, 0)),
        ],
        out_shape=[
            jax.ShapeDtypeStruct((n_main, 1, _G), jnp.int32),
            jax.ShapeDtypeStruct((n_main, 1, _G), jnp.float32),
        ],
        compiler_params=pltpu.CompilerParams(
            dimension_semantics=("arbitrary",)
        ),
        interpret=interpret,
    )(hm)

    idx_a, logp_a = pl.pallas_call(
        _assist_body,
        grid=(_X_SC // _ROWS,),
        in_specs=[
            pl.BlockSpec((_ROWS, _H, _W), lambda r: (r, 0, 0)),
            pl.BlockSpec((_ROWS, _G, _H, _W), lambda r: (r, 0, 0, 0)),
        ],
        out_specs=[
            pl.BlockSpec((_ROWS, 1, _G), lambda r: (r, 0, 0)),
            pl.BlockSpec((_ROWS, 1, _G), lambda r: (r, 0, 0)),
        ],
        out_shape=[
            jax.ShapeDtypeStruct((_X_SC, 1, _G), jnp.int32),
            jax.ShapeDtypeStruct((_X_SC, 1, _G), jnp.float32),
        ],
        compiler_params=pltpu.CompilerParams(
            dimension_semantics=("arbitrary",)
        ),
        interpret=interpret,
    )(hm[:_X_SC], u_sc)

    idx = jnp.concatenate([idx_a, idx_m], axis=0).reshape(_B, _K, _G)
    logp = jnp.concatenate([logp_a, logp_m], axis=0).reshape(_B, _K, _G)
    out = pl.pallas_call(
        _loss_body,
        in_specs=[
            pl.BlockSpec(idx.shape, lambda: (0, 0, 0)),
            pl.BlockSpec(logp.shape, lambda: (0, 0, 0)),
        ],
        out_specs=pl.BlockSpec((1, 128), lambda: (0, 0)),
        out_shape=jax.ShapeDtypeStruct((1, 128), jnp.float32),
        interpret=interpret,
    )(idx, logp)
    return (out[0, 0], out[0, 1], out[0, 2], out[0, 3])


def kernel(heatmaps):
    return _run(heatmaps)


# speedup vs baseline: 1.2015x; 1.0069x over previous
import functools

import jax
import jax.numpy as jnp
import numpy as np
from jax import lax
from jax.experimental import pallas as pl
from jax.experimental.pallas import tpu as pltpu
from jax.experimental.pallas import tpu_sc as plsc

_B, _K, _H, _W = 64, 17, 128, 128
_V = _H * _W
_G = 8
_R = _B * _K
_ROWS = 8
_NW = 32
_X_SC = 328

_TINY = np.float32(np.finfo(np.float32).tiny)
_EPS = np.float32(1e-8)
_K1 = np.int32(42)
_K2 = np.int32(0 ^ 42 ^ 0x1BD11BDA)
_ROT_A = (13, 15, 26, 6)
_ROT_B = (17, 29, 16, 24)


def _rotl(x, d):
    return lax.shift_left(x, np.int32(d)) | lax.shift_right_logical(
        x, np.int32(32 - d)
    )


def _four_rounds(x0, x1, rots):
    for r in rots:
        x0 = x0 + x1
        x1 = _rotl(x1, r)
        x1 = x0 ^ x1
    return x0, x1


def _threefry_bits(x1):
    x0 = x1
    x1 = x0 ^ _rotl(x1, _ROT_A[0])
    for r in _ROT_A[1:]:
        x0 = x0 + x1
        x1 = _rotl(x1, r)
        x1 = x0 ^ x1
    x0 = x0 + _K1
    x1 = x1 + np.int32(_K2 + 1)
    x0, x1 = _four_rounds(x0, x1, _ROT_B)
    x0 = x0 + _K2
    x1 = x1 + np.int32(2)
    x0, x1 = _four_rounds(x0, x1, _ROT_A)
    x1 = x1 + np.int32(_K1 + 3)
    x0, x1 = _four_rounds(x0, x1, _ROT_B)
    x0 = x0 + _K1
    x1 = x1 + np.int32(_K2 + 4)
    x0, x1 = _four_rounds(x0, x1, _ROT_A)
    x0 = x0 + _K2
    x1 = x1 + np.int32(5)
    return x0 ^ x1


def _bits_to_u(bits):
    fbits = lax.shift_right_logical(bits, np.int32(9)) | np.int32(0x3F800000)
    f = lax.bitcast_convert_type(fbits, jnp.float32) - np.float32(1.0)
    return jnp.maximum(_TINY, f + _TINY)


def _sc_body(out_hbm, buf, sem):
    c = lax.axis_index("c")
    s_ax = lax.axis_index("s")
    wid = s_ax * np.int32(2) + c
    ppw = np.int32(_X_SC * _G // _NW)
    lane = lax.iota(jnp.int32, 16)

    def pair_body(i, carry):
        pair = wid * ppw + i
        row = pair // np.int32(_G)
        s = pair % np.int32(_G)
        base = (s * np.int32(_R) + row) * np.int32(_V) + np.int32(42)

        def chunk_body(cc, carry2):
            h = cc
            for q in range(8):
                p = base + h * np.int32(_W) + np.int32(q * 16) + lane
                buf[h, pl.ds(q * 16, 16)] = _bits_to_u(_threefry_bits(p))
            return carry2

        lax.fori_loop(0, _H, chunk_body, 0, unroll=2)
        pltpu.sync_copy(buf, out_hbm.at[row, s])
        return carry

    lax.fori_loop(0, ppw, pair_body, 0, unroll=False)


def _sc_uniforms():
    mesh = plsc.VectorSubcoreMesh(core_axis_name="c", subcore_axis_name="s")
    return pl.kernel(
        _sc_body,
        mesh=mesh,
        out_type=jax.ShapeDtypeStruct((_X_SC, _G, _H, _W), jnp.float32),
        scratch_types=[
            pltpu.VMEM((_H, _W), jnp.float32),
            pltpu.SemaphoreType.DMA,
        ],
    )()


def _extract(l, vi, z, m, lse):
    zm = jnp.max(z)
    win = jnp.min(jnp.where(z == zm, vi, np.int32(_V)))
    lwin = jnp.sum(jnp.where(vi == win, l, np.float32(0.0)))
    return win, (lwin - m) - lse


def _row_outputs(wins, logps, rr, sub, lane, idx_out, logp_out):
    for s in range(_G):
        here = (sub == rr) & (lane == s)
        idx_out = jnp.where(here, wins[s], idx_out)
        logp_out = jnp.where(here, logps[s], logp_out)
    return idx_out, logp_out


def _sample_body(hm_ref, idx_ref, logp_ref, *, row_offset):
    rb = pl.program_id(0)
    vi = (
        lax.broadcasted_iota(jnp.int32, (_H, _W), 0) * np.int32(_W)
        + lax.broadcasted_iota(jnp.int32, (_H, _W), 1)
    )
    sub = lax.broadcasted_iota(jnp.int32, (_ROWS, 1, _G), 0)
    lane = lax.broadcasted_iota(jnp.int32, (_ROWS, 1, _G), 2)
    idx_out = jnp.zeros((_ROWS, 1, _G), jnp.int32)
    logp_out = jnp.zeros((_ROWS, 1, _G), jnp.float32)

    for rr in range(_ROWS):
        r = rb * np.int32(_ROWS) + np.int32(row_offset + rr)
        l = hm_ref[rr]
        m = jnp.max(l)
        lse = jnp.log(jnp.sum(jnp.exp(l - m)))
        zs = []
        for s in range(_G):
            base = (np.int32(s * _R) + r) * np.int32(_V) + np.int32(42)
            u = _bits_to_u(_threefry_bits(base + vi))
            zs.append(-jnp.log(-jnp.log(u)) + l)
        zms = [jnp.max(z) for z in zs]
        wins = []
        logps = []
        for s in range(_G):
            win = jnp.min(jnp.where(zs[s] == zms[s], vi, np.int32(_V)))
            wins.append(win)
        for s in range(_G):
            lwin = jnp.sum(jnp.where(vi == wins[s], l, np.float32(0.0)))
            logps.append((lwin - m) - lse)
        idx_out, logp_out = _row_outputs(
            wins, logps, rr, sub, lane, idx_out, logp_out
        )
    idx_ref[...] = idx_out
    logp_ref[...] = logp_out


def _assist_body(hm_ref, u_ref, idx_ref, logp_ref):
    vi = (
        lax.broadcasted_iota(jnp.int32, (_H, _W), 0) * np.int32(_W)
        + lax.broadcasted_iota(jnp.int32, (_H, _W), 1)
    )
    sub = lax.broadcasted_iota(jnp.int32, (_ROWS, 1, _G), 0)
    lane = lax.broadcasted_iota(jnp.int32, (_ROWS, 1, _G), 2)
    idx_out = jnp.zeros((_ROWS, 1, _G), jnp.int32)
    logp_out = jnp.zeros((_ROWS, 1, _G), jnp.float32)

    for rr in range(_ROWS):
        l = hm_ref[rr]
        m = jnp.max(l)
        lse = jnp.log(jnp.sum(jnp.exp(l - m)))
        zs = [
            -jnp.log(-jnp.log(u_ref[rr, s])) + l for s in range(_G)
        ]
        zms = [jnp.max(z) for z in zs]
        wins = []
        logps = []
        for s in range(_G):
            win = jnp.min(jnp.where(zs[s] == zms[s], vi, np.int32(_V)))
            wins.append(win)
        for s in range(_G):
            lwin = jnp.sum(jnp.where(vi == wins[s], l, np.float32(0.0)))
            logps.append((lwin - m) - lse)
        idx_out, logp_out = _row_outputs(
            wins, logps, rr, sub, lane, idx_out, logp_out
        )
    idx_ref[...] = idx_out
    logp_ref[...] = logp_out


def _loss_body(idx_ref, logp_ref, out_ref):
    idx = idx_ref[...]
    logp = logp_ref[...]
    x = (idx % np.int32(_W)).astype(jnp.float32)
    y = (idx // np.int32(_W)).astype(jnp.float32)
    cx = np.float32((_W - 1) / 2.0)
    cy = np.float32((_H - 1) / 2.0)
    d = jnp.sqrt((x - cx) * (x - cx) + (y - cy) * (y - cy))
    rewards = -(jnp.sum(d, axis=1) / np.float32(_K)) / np.float32(max(_H, _W))
    rmean = jnp.mean(rewards, axis=-1, keepdims=True)
    dev = rewards - rmean
    std = jnp.sqrt(jnp.sum(dev * dev, axis=-1, keepdims=True) / np.float32(_G - 1))
    adv = dev / jnp.maximum(std, _EPS)
    adv = jnp.clip(adv, -5.0, 5.0)
    log_pi = jnp.sum(logp, axis=1)
    loss = -jnp.mean(adv * log_pi)
    reward_mean = jnp.mean(rewards)
    rdev = rewards - reward_mean
    reward_std = jnp.sqrt(jnp.sum(rdev * rdev) / np.float32(_B * _G - 1))
    adv_abs_mean = jnp.mean(jnp.abs(adv))
    lanes = lax.broadcasted_iota(jnp.int32, (1, 128), 1)
    vec = jnp.where(lanes == 0, loss, np.float32(0.0))
    vec = jnp.where(lanes == 1, reward_mean, vec)
    vec = jnp.where(lanes == 2, reward_std, vec)
    vec = jnp.where(lanes == 3, adv_abs_mean, vec)
    out_ref[...] = vec


def _run(heatmaps, interpret=False):
    hm = heatmaps.reshape(_R, _H, _W)

    u_sc = _sc_uniforms()

    n_main = _R - _X_SC
    idx_m, logp_m = pl.pallas_call(
        functools.partial(_sample_body, row_offset=_X_SC),
        grid=(n_main // _ROWS,),
        in_specs=[
            pl.BlockSpec((_ROWS, _H, _W), lambda r: (r + _X_SC // _ROWS, 0, 0)),
        ],
        out_specs=[
            pl.BlockSpec((_ROWS, 1, _G), lambda r: (r, 0, 0)),
            pl.BlockSpec((_ROWS, 1, _G), lambda r: (r, 0, 0)),
        ],
        out_shape=[
            jax.ShapeDtypeStruct((n_main, 1, _G), jnp.int32),
            jax.ShapeDtypeStruct((n_main, 1, _G), jnp.float32),
        ],
        compiler_params=pltpu.CompilerParams(
            dimension_semantics=("arbitrary",)
        ),
        interpret=interpret,
    )(hm)

    idx_a, logp_a = pl.pallas_call(
        _assist_body,
        grid=(_X_SC // _ROWS,),
        in_specs=[
            pl.BlockSpec((_ROWS, _H, _W), lambda r: (r, 0, 0)),
            pl.BlockSpec((_ROWS, _G, _H, _W), lambda r: (r, 0, 0, 0)),
        ],
        out_specs=[
            pl.BlockSpec((_ROWS, 1, _G), lambda r: (r, 0, 0)),
            pl.BlockSpec((_ROWS, 1, _G), lambda r: (r, 0, 0)),
        ],
        out_shape=[
            jax.ShapeDtypeStruct((_X_SC, 1, _G), jnp.int32),
            jax.ShapeDtypeStruct((_X_SC, 1, _G), jnp.float32),
        ],
        compiler_params=pltpu.CompilerParams(
            dimension_semantics=("arbitrary",)
        ),
        interpret=interpret,
    )(hm[:_X_SC], u_sc)

    idx = jnp.concatenate([idx_a, idx_m], axis=0).reshape(_B, _K, _G)
    logp = jnp.concatenate([logp_a, logp_m], axis=0).reshape(_B, _K, _G)
    out = pl.pallas_call(
        _loss_body,
        in_specs=[
            pl.BlockSpec(idx.shape, lambda: (0, 0, 0)),
            pl.BlockSpec(logp.shape, lambda: (0, 0, 0)),
        ],
        out_specs=pl.BlockSpec((1, 128), lambda: (0, 0)),
        out_shape=jax.ShapeDtypeStruct((1, 128), jnp.float32),
        interpret=interpret,
    )(idx, logp)
    return (out[0, 0], out[0, 1], out[0, 2], out[0, 3])


def kernel(heatmaps):
    return _run(heatmaps)
